# baseline (speedup 1.0000x reference)
import functools

import jax
import jax.numpy as jnp
from jax import lax
from jax.experimental import pallas as pl
from jax.experimental.pallas import tpu as pltpu
from jax.experimental.pallas import tpu_sc as plsc

BN = 512
NP = 100352
NC = 2
NS = 16
NW = NC * NS
B_PER_W = NP // NW
KCH = 784


def _k1_body(x_ref, ws_ref, wn_ref, t1_ref, t2_ref, u_ref):
    xb = x_ref[...]
    ws = ws_ref[...]
    wn = wn_ref[...]
    a = lax.dot_general(xb, ws, (((1,), (0,)), ((), ())),
                        preferred_element_type=jnp.float32)
    c = lax.dot_general(xb, wn, (((1,), (0,)), ((), ())),
                        preferred_element_type=jnp.float32)
    x3 = xb[:, 0:3]
    p = (x3[:, 0:1] * wn[0:1, :] + x3[:, 1:2] * wn[1:2, :]
         + x3[:, 2:3] * wn[2:3, :])
    b = c - p
    z13 = jnp.zeros((xb.shape[0], 13), jnp.float32)
    t1_ref[...] = c
    t2_ref[...] = jnp.concatenate([x3, z13], axis=1)
    u_ref[...] = jnp.concatenate([a, b, x3, z13], axis=1)


def _sc_gather_body(t1_hbm, t2_hbm, e_hbm, g1_hbm, g2_hbm,
                    idx_v, r1_v, r2_v, sem1, sem2):
    wid = lax.axis_index("s") * NC + lax.axis_index("c")
    base = wid * B_PER_W
    for j in range(B_PER_W // KCH):
        off = base + j * KCH
        pltpu.sync_copy(e_hbm.at[pl.ds(off, KCH)], idx_v)
        c1 = pltpu.async_copy(t1_hbm.at[idx_v], r1_v, sem1)
        c2 = pltpu.async_copy(t2_hbm.at[idx_v], r2_v, sem2)
        c1.wait()
        c2.wait()
        pltpu.sync_copy(r1_v, g1_hbm.at[pl.ds(off, KCH)])
        pltpu.sync_copy(r2_v, g2_hbm.at[pl.ds(off, KCH)])


def _k3_body(g1_ref, g2_ref, u_ref, vm_ref, out_ref):
    g1 = g1_ref[...]
    n3 = g2_ref[:, 0:3]
    u = u_ref[...]
    a = u[:, 0:32]
    b = u[:, 32:64]
    x3 = u[:, 64:67]
    vm = vm_ref[...]
    diff = x3 - n3
    d2 = jnp.sum(diff * diff, axis=1, keepdims=True)
    d2c = jnp.where(d2 > 0.0, d2, jnp.float32(1e-4))
    out_ref[...] = ((g1 + b) / d2c + a) * vm


def kernel(x, neighbor_idx, w_s, w_n):
    n, f = x.shape
    c = w_s.shape[1]
    grid = NP // BN

    idx0 = neighbor_idx[:, 0]
    idx1 = neighbor_idx[:, 1]
    e = jnp.where(idx1 != 0, idx1, idx0)
    e_pad = jnp.concatenate([e, jnp.zeros((NP - n,), jnp.int32)])
    vmask = (e != 0).astype(jnp.float32)[:, None]

    t1, t2, u = pl.pallas_call(
        _k1_body,
        grid=(grid,),
        in_specs=[
            pl.BlockSpec((BN, f), lambda i: (i, 0)),
            pl.BlockSpec((f, c), lambda i: (0, 0)),
            pl.BlockSpec((f, c), lambda i: (0, 0)),
        ],
        out_specs=[
            pl.BlockSpec((BN, 32), lambda i: (i, 0)),
            pl.BlockSpec((BN, 16), lambda i: (i, 0)),
            pl.BlockSpec((BN, 80), lambda i: (i, 0)),
        ],
        out_shape=[
            jax.ShapeDtypeStruct((NP, 32), jnp.float32),
            jax.ShapeDtypeStruct((NP, 16), jnp.float32),
            jax.ShapeDtypeStruct((NP, 80), jnp.float32),
        ],
    )(x, w_s, w_n)

    mesh = plsc.VectorSubcoreMesh(core_axis_name="c", subcore_axis_name="s",
                                  num_cores=NC, num_subcores=NS)
    g1, g2 = pl.kernel(
        _sc_gather_body,
        out_type=[
            jax.ShapeDtypeStruct((NP, 32), jnp.float32),
            jax.ShapeDtypeStruct((NP, 16), jnp.float32),
        ],
        mesh=mesh,
        scratch_types=[
            pltpu.VMEM((KCH,), jnp.int32),
            pltpu.VMEM((KCH, 32), jnp.float32),
            pltpu.VMEM((KCH, 16), jnp.float32),
            pltpu.SemaphoreType.DMA,
            pltpu.SemaphoreType.DMA,
        ],
        compiler_params=pltpu.CompilerParams(use_tc_tiling_on_sc=False),
    )(t1, t2, e_pad)

    out = pl.pallas_call(
        _k3_body,
        grid=(grid,),
        in_specs=[
            pl.BlockSpec((BN, 32), lambda i: (i, 0)),
            pl.BlockSpec((BN, 16), lambda i: (i, 0)),
            pl.BlockSpec((BN, 80), lambda i: (i, 0)),
            pl.BlockSpec((BN, 1), lambda i: (i, 0)),
        ],
        out_specs=pl.BlockSpec((BN, 32), lambda i: (i, 0)),
        out_shape=jax.ShapeDtypeStruct((n, 32), jnp.float32),
    )(g1, g2, u, vmask)
    return out


# speedup vs baseline: 1.0007x; 1.0007x over previous
import jax
import jax.numpy as jnp
from jax import lax
from jax.experimental import pallas as pl
from jax.experimental.pallas import tpu as pltpu
from jax.experimental.pallas import tpu_sc as plsc

BN = 512
NP = 100352
NC = 2
NS = 16
NW = NC * NS
B_PER_W = NP // NW
KCH = 784
NCH = B_PER_W // KCH


def _k1_body(x_ref, ws_ref, wn_ref, t1_ref, t2_ref, a_ref, b_ref):
    xb = x_ref[...]
    ws = ws_ref[...]
    wn = wn_ref[...]
    lane = lax.broadcasted_iota(jnp.int32, xb.shape, 1)
    xt = jnp.where(lane < 3, jnp.float32(0.0), xb)
    dn = (((1,), (0,)), ((), ()))
    a_ref[...] = lax.dot_general(xb, ws, dn, preferred_element_type=jnp.float32)
    t1_ref[...] = lax.dot_general(xb, wn, dn, preferred_element_type=jnp.float32)
    b_ref[...] = lax.dot_general(xt, wn, dn, preferred_element_type=jnp.float32)
    x16 = x_ref[:, 0:16]
    lane16 = lax.broadcasted_iota(jnp.int32, x16.shape, 1)
    t2_ref[...] = jnp.where(lane16 < 3, x16, jnp.float32(0.0))


def _sc_gather_body(t1_hbm, t2_hbm, e_hbm, g1_hbm, g2_hbm,
                    idx_v, r1_v, r2_v, gsem1, gsem2, wsem):
    wid = lax.axis_index("s") * NC + lax.axis_index("c")
    base = wid * B_PER_W
    pltpu.sync_copy(e_hbm.at[pl.ds(base, B_PER_W)], idx_v)

    def start_gather(j):
        b = j % 2
        i = idx_v.at[pl.ds(j * KCH, KCH)]
        c1 = pltpu.async_copy(t1_hbm.at[i], r1_v.at[b], gsem1)
        c2 = pltpu.async_copy(t2_hbm.at[i], r2_v.at[b], gsem2)
        return c1, c2

    def start_writeback(j):
        b = j % 2
        w1 = pltpu.async_copy(r1_v.at[b], g1_hbm.at[pl.ds(base + j * KCH, KCH)],
                              wsem)
        w2 = pltpu.async_copy(r2_v.at[b], g2_hbm.at[pl.ds(base + j * KCH, KCH)],
                              wsem)
        return w1, w2

    g_cp = {0: start_gather(0)}
    w_cp = {}
    for j in range(NCH):
        if j + 1 < NCH:
            if j - 1 in w_cp:
                for w in w_cp.pop(j - 1):
                    w.wait()
            g_cp[j + 1] = start_gather(j + 1)
        for g in g_cp.pop(j):
            g.wait()
        w_cp[j] = start_writeback(j)
    for j in sorted(w_cp):
        for w in w_cp[j]:
            w.wait()


def _k3_body(g1_ref, g2_ref, t2_ref, a_ref, b_ref, vm_ref, out_ref):
    g1 = g1_ref[...]
    diff = t2_ref[...] - g2_ref[...]
    d2 = jnp.sum(diff * diff, axis=1, keepdims=True)
    rec = 1.0 / jnp.where(d2 > 0.0, d2, jnp.float32(1e-4))
    out_ref[...] = ((g1 + b_ref[...]) * rec + a_ref[...]) * vm_ref[...]


def kernel(x, neighbor_idx, w_s, w_n):
    n, f = x.shape
    c = w_s.shape[1]
    grid = NP // BN

    idx0 = neighbor_idx[:, 0]
    idx1 = neighbor_idx[:, 1]
    e = jnp.where(idx1 != 0, idx1, idx0)
    e_pad = jnp.concatenate([e, jnp.zeros((NP - n,), jnp.int32)])
    vmask = (e != 0).astype(jnp.float32)[:, None]

    t1, t2, a, b = pl.pallas_call(
        _k1_body,
        grid=(grid,),
        in_specs=[
            pl.BlockSpec((BN, f), lambda i: (i, 0)),
            pl.BlockSpec((f, c), lambda i: (0, 0)),
            pl.BlockSpec((f, c), lambda i: (0, 0)),
        ],
        out_specs=[
            pl.BlockSpec((BN, 32), lambda i: (i, 0)),
            pl.BlockSpec((BN, 16), lambda i: (i, 0)),
            pl.BlockSpec((BN, 32), lambda i: (i, 0)),
            pl.BlockSpec((BN, 32), lambda i: (i, 0)),
        ],
        out_shape=[
            jax.ShapeDtypeStruct((NP, 32), jnp.float32),
            jax.ShapeDtypeStruct((NP, 16), jnp.float32),
            jax.ShapeDtypeStruct((NP, 32), jnp.float32),
            jax.ShapeDtypeStruct((NP, 32), jnp.float32),
        ],
    )(x, w_s, w_n)

    mesh = plsc.VectorSubcoreMesh(core_axis_name="c", subcore_axis_name="s",
                                  num_cores=NC, num_subcores=NS)
    g1, g2 = pl.kernel(
        _sc_gather_body,
        out_type=[
            jax.ShapeDtypeStruct((NP, 32), jnp.float32),
            jax.ShapeDtypeStruct((NP, 16), jnp.float32),
        ],
        mesh=mesh,
        scratch_types=[
            pltpu.VMEM((B_PER_W,), jnp.int32),
            pltpu.VMEM((2, KCH, 32), jnp.float32),
            pltpu.VMEM((2, KCH, 16), jnp.float32),
            pltpu.SemaphoreType.DMA,
            pltpu.SemaphoreType.DMA,
            pltpu.SemaphoreType.DMA,
        ],
        compiler_params=pltpu.CompilerParams(use_tc_tiling_on_sc=False),
    )(t1, t2, e_pad)

    out = pl.pallas_call(
        _k3_body,
        grid=(grid,),
        in_specs=[
            pl.BlockSpec((BN, 32), lambda i: (i, 0)),
            pl.BlockSpec((BN, 16), lambda i: (i, 0)),
            pl.BlockSpec((BN, 16), lambda i: (i, 0)),
            pl.BlockSpec((BN, 32), lambda i: (i, 0)),
            pl.BlockSpec((BN, 32), lambda i: (i, 0)),
            pl.BlockSpec((BN, 1), lambda i: (i, 0)),
        ],
        out_specs=pl.BlockSpec((BN, 32), lambda i: (i, 0)),
        out_shape=jax.ShapeDtypeStruct((n, 32), jnp.float32),
    )(g1, g2, t2, a, b, vmask)
    return out


# speedup vs baseline: 2.2061x; 2.2046x over previous
import jax
import jax.numpy as jnp
from jax import lax
from jax.experimental import pallas as pl
from jax.experimental.pallas import tpu as pltpu
from jax.experimental.pallas import tpu_sc as plsc

BN = 512
NP = 100352
NC = 2
NS = 16
NW = NC * NS
B_PER_W = NP // NW
KCH = 448
NCH = B_PER_W // KCH


def _sc_gather_body(x_hbm, e_hbm, g_hbm, idx_v, r_v, gsem, wsem):
    wid = lax.axis_index("s") * NC + lax.axis_index("c")
    base = wid * B_PER_W
    pltpu.sync_copy(e_hbm.at[pl.ds(base, B_PER_W)], idx_v)

    def start_gather(j):
        return pltpu.async_copy(
            x_hbm.at[idx_v.at[pl.ds(j * KCH, KCH)]], r_v.at[j % 2], gsem)

    def start_writeback(j):
        return pltpu.async_copy(
            r_v.at[j % 2], g_hbm.at[pl.ds(base + j * KCH, KCH)], wsem)

    g_cp = {0: start_gather(0)}
    w_cp = {}
    for j in range(NCH):
        if j + 1 < NCH:
            if j - 1 in w_cp:
                w_cp.pop(j - 1).wait()
            g_cp[j + 1] = start_gather(j + 1)
        g_cp.pop(j).wait()
        w_cp[j] = start_writeback(j)
    for j in sorted(w_cp):
        w_cp[j].wait()


def _combine_body(x_ref, g_ref, ws_ref, wn_ref, out_ref):
    xb = x_ref[...]
    g = g_ref[...]
    lane = lax.broadcasted_iota(jnp.int32, xb.shape, 1)
    first3 = lane < 3
    comb = jnp.where(first3, g, g + xb)
    diff = jnp.where(first3, xb - g, jnp.float32(0.0))
    d2 = jnp.sum(diff * diff, axis=1, keepdims=True)
    rec = 1.0 / jnp.where(d2 > 0.0, d2, jnp.float32(1e-4))
    dn = (((1,), (0,)), ((), ()))
    r = lax.dot_general(comb, wn_ref[...], dn,
                        preferred_element_type=jnp.float32)
    s = lax.dot_general(xb, ws_ref[...], dn,
                        preferred_element_type=jnp.float32)
    out_ref[...] = r * rec + s


def kernel(x, neighbor_idx, w_s, w_n):
    n, f = x.shape
    c = w_s.shape[1]
    grid = NP // BN

    idx0 = neighbor_idx[:, 0]
    idx1 = neighbor_idx[:, 1]
    e = jnp.where(idx1 != 0, idx1, idx0)
    e_pad = jnp.concatenate([e, jnp.zeros((NP - n,), jnp.int32)])

    mesh = plsc.VectorSubcoreMesh(core_axis_name="c", subcore_axis_name="s",
                                  num_cores=NC, num_subcores=NS)
    g = pl.kernel(
        _sc_gather_body,
        out_type=jax.ShapeDtypeStruct((NP, f), jnp.float32),
        mesh=mesh,
        scratch_types=[
            pltpu.VMEM((B_PER_W,), jnp.int32),
            pltpu.VMEM((2, KCH, f), jnp.float32),
            pltpu.SemaphoreType.DMA,
            pltpu.SemaphoreType.DMA,
        ],
    )(x, e_pad)

    out = pl.pallas_call(
        _combine_body,
        grid=(grid,),
        in_specs=[
            pl.BlockSpec((BN, f), lambda i: (i, 0)),
            pl.BlockSpec((BN, f), lambda i: (i, 0)),
            pl.BlockSpec((f, c), lambda i: (0, 0)),
            pl.BlockSpec((f, c), lambda i: (0, 0)),
        ],
        out_specs=pl.BlockSpec((BN, 32), lambda i: (i, 0)),
        out_shape=jax.ShapeDtypeStruct((n, 32), jnp.float32),
    )(x, g, w_s, w_n)

    return out * (e != 0).astype(jnp.float32)[:, None]


# speedup vs baseline: 2.8051x; 1.2715x over previous
import jax
import jax.numpy as jnp
from jax import lax
from jax.experimental import pallas as pl
from jax.experimental.pallas import tpu as pltpu
from jax.experimental.pallas import tpu_sc as plsc

BN = 512
NP = 100352
NC = 2
NS = 16
NW = NC * NS
B_PER_W = NP // NW
KCH = 448
NCH = B_PER_W // KCH


def _sc_gather_body(x_hbm, e_hbm, g_hbm, idx_v, r_v, gsem, wsem):
    wid = lax.axis_index("s") * NC + lax.axis_index("c")
    base = wid * B_PER_W
    pltpu.sync_copy(e_hbm.at[pl.ds(base, B_PER_W)], idx_v)

    def start_gather(j):
        return pltpu.async_copy(
            x_hbm.at[idx_v.at[pl.ds(j * KCH, KCH)]], r_v.at[j % 2], gsem)

    def start_writeback(j):
        return pltpu.async_copy(
            r_v.at[j % 2], g_hbm.at[pl.ds(base + j * KCH, KCH)], wsem)

    g_cp = {0: start_gather(0)}
    w_cp = {}
    for j in range(NCH):
        if j + 1 < NCH:
            if j - 1 in w_cp:
                w_cp.pop(j - 1).wait()
            g_cp[j + 1] = start_gather(j + 1)
        g_cp.pop(j).wait()
        w_cp[j] = start_writeback(j)
    for j in sorted(w_cp):
        w_cp[j].wait()


def _combine_body(x_ref, g_ref, ws_ref, wn_ref, out_ref):
    xb = x_ref[...]
    g = g_ref[...]
    lane = lax.broadcasted_iota(jnp.int32, xb.shape, 1)
    first3 = lane < 3
    comb = jnp.where(first3, g, g + xb)
    diff = jnp.where(first3, xb - g, jnp.float32(0.0))
    ones = jnp.ones((1, xb.shape[1]), jnp.float32)
    dt = (((1,), (1,)), ((), ()))
    d2 = lax.dot_general(ones, diff * diff, dt,
                         preferred_element_type=jnp.float32)
    rec = 1.0 / jnp.where(d2 > 0.0, d2, jnp.float32(1e-4))
    dn = (((0,), (1,)), ((), ()))
    r = lax.dot_general(wn_ref[...], comb, dn,
                        preferred_element_type=jnp.float32)
    s = lax.dot_general(ws_ref[...], xb, dn,
                        preferred_element_type=jnp.float32)
    out_ref[...] = r * rec + s


def kernel(x, neighbor_idx, w_s, w_n):
    n, f = x.shape
    c = w_s.shape[1]
    grid = NP // BN

    idx0 = neighbor_idx[:, 0]
    idx1 = neighbor_idx[:, 1]
    e = jnp.where(idx1 != 0, idx1, idx0)
    e_pad = jnp.concatenate([e, jnp.zeros((NP - n,), jnp.int32)])

    mesh = plsc.VectorSubcoreMesh(core_axis_name="c", subcore_axis_name="s",
                                  num_cores=NC, num_subcores=NS)
    g = pl.kernel(
        _sc_gather_body,
        out_type=jax.ShapeDtypeStruct((NP, f), jnp.float32),
        mesh=mesh,
        scratch_types=[
            pltpu.VMEM((B_PER_W,), jnp.int32),
            pltpu.VMEM((2, KCH, f), jnp.float32),
            pltpu.SemaphoreType.DMA,
            pltpu.SemaphoreType.DMA,
        ],
    )(x, e_pad)

    out_t = pl.pallas_call(
        _combine_body,
        grid=(grid,),
        in_specs=[
            pl.BlockSpec((BN, f), lambda i: (i, 0)),
            pl.BlockSpec((BN, f), lambda i: (i, 0)),
            pl.BlockSpec((f, c), lambda i: (0, 0)),
            pl.BlockSpec((f, c), lambda i: (0, 0)),
        ],
        out_specs=pl.BlockSpec((c, BN), lambda i: (0, i)),
        out_shape=jax.ShapeDtypeStruct((c, n), jnp.float32),
    )(x, g, w_s, w_n)

    return (out_t * (e != 0).astype(jnp.float32)[None, :]).T


# speedup vs baseline: 4.2622x; 1.5194x over previous
import jax
import jax.numpy as jnp
from jax import lax
from jax.experimental import pallas as pl
from jax.experimental.pallas import tpu as pltpu
from jax.experimental.pallas import tpu_sc as plsc

BN = 2048
NP = 100352
NC = 2
NS = 16
NW = NC * NS
B_PER_W = NP // NW
KCH = 448
NCH = B_PER_W // KCH


def _sc_gather_body(x_hbm, e_hbm, g_hbm, idx_v, r_v, gsem, wsem):
    wid = lax.axis_index("s") * NC + lax.axis_index("c")
    base = wid * B_PER_W
    pltpu.sync_copy(e_hbm.at[pl.ds(base, B_PER_W)], idx_v)

    def start_gather(j):
        return pltpu.async_copy(
            x_hbm.at[idx_v.at[pl.ds(j * KCH, KCH)]], r_v.at[j % 2], gsem)

    def start_writeback(j):
        return pltpu.async_copy(
            r_v.at[j % 2], g_hbm.at[pl.ds(base + j * KCH, KCH)], wsem)

    g_cp = {0: start_gather(0)}
    w_cp = {}
    for j in range(NCH):
        if j + 1 < NCH:
            if j - 1 in w_cp:
                w_cp.pop(j - 1).wait()
            g_cp[j + 1] = start_gather(j + 1)
        g_cp.pop(j).wait()
        w_cp[j] = start_writeback(j)
    for j in sorted(w_cp):
        w_cp[j].wait()


def _combine_body(x_ref, g_ref, ws_ref, wn_ref, out_ref):
    xb = x_ref[...]
    g = g_ref[...]
    lane = lax.broadcasted_iota(jnp.int32, xb.shape, 1)
    first3 = lane < 3
    comb = jnp.where(first3, g, g + xb)
    diff = jnp.where(first3, xb - g, jnp.float32(0.0))
    ones = jnp.ones((1, xb.shape[1]), jnp.float32)
    dt = (((1,), (1,)), ((), ()))
    d2 = lax.dot_general(ones, diff * diff, dt,
                         preferred_element_type=jnp.float32)
    rec = 1.0 / jnp.where(d2 > 0.0, d2, jnp.float32(1e-4))
    dn = (((0,), (1,)), ((), ()))
    r = lax.dot_general(wn_ref[...], comb, dn,
                        preferred_element_type=jnp.float32)
    s = lax.dot_general(ws_ref[...], xb, dn,
                        preferred_element_type=jnp.float32)
    out_ref[...] = r * rec + s


def kernel(x, neighbor_idx, w_s, w_n):
    n, f = x.shape
    c = w_s.shape[1]
    grid = NP // BN

    idx0 = neighbor_idx[:, 0]
    idx1 = neighbor_idx[:, 1]
    e = jnp.where(idx1 != 0, idx1, idx0)
    e_pad = jnp.concatenate([e, jnp.zeros((NP - n,), jnp.int32)])

    mesh = plsc.VectorSubcoreMesh(core_axis_name="c", subcore_axis_name="s",
                                  num_cores=NC, num_subcores=NS)
    g = pl.kernel(
        _sc_gather_body,
        out_type=jax.ShapeDtypeStruct((NP, f), jnp.float32),
        mesh=mesh,
        scratch_types=[
            pltpu.VMEM((B_PER_W,), jnp.int32),
            pltpu.VMEM((2, KCH, f), jnp.float32),
            pltpu.SemaphoreType.DMA,
            pltpu.SemaphoreType.DMA,
        ],
    )(x, e_pad)

    out_t = pl.pallas_call(
        _combine_body,
        grid=(grid,),
        in_specs=[
            pl.BlockSpec((BN, f), lambda i: (i, 0)),
            pl.BlockSpec((BN, f), lambda i: (i, 0)),
            pl.BlockSpec((f, c), lambda i: (0, 0)),
            pl.BlockSpec((f, c), lambda i: (0, 0)),
        ],
        out_specs=pl.BlockSpec((c, BN), lambda i: (0, i)),
        out_shape=jax.ShapeDtypeStruct((c, n), jnp.float32),
    )(x, g, w_s, w_n)

    return (out_t * (e != 0).astype(jnp.float32)[None, :]).T
